# speedup vs baseline: 1.3923x; 1.3923x over previous
import functools

import jax
import jax.numpy as jnp
from jax.experimental import pallas as pl
from jax.experimental.pallas import tpu as pltpu

CHUNK = 1024


def _rpq_kernel(x_ref, p_ref, ct_ref, csq_ref, out_ref):
    x = x_ref[0]
    t = jnp.dot(x, p_ref[...], preferred_element_type=jnp.float32)
    nrm = jnp.sqrt(jnp.sum(t * t, axis=-1, keepdims=True))
    tn = t / jnp.maximum(nrm, 1e-12)
    tsq = jnp.sum(tn * tn, axis=-1, keepdims=True)

    L = x.shape[0]
    K = ct_ref.shape[1]

    def body(i, carry):
        best_val, best_idx = carry
        ct = ct_ref[:, pl.ds(i * CHUNK, CHUNK)]
        csq = csq_ref[:, pl.ds(i * CHUNK, CHUNK)]
        cross = jnp.dot(tn, ct, preferred_element_type=jnp.float32)
        d2 = jnp.maximum(tsq + csq - 2.0 * cross, 0.0)
        cmin = jnp.min(d2, axis=1, keepdims=True)
        idx = jax.lax.broadcasted_iota(jnp.int32, d2.shape, 1) + i * CHUNK
        carg = jnp.min(jnp.where(d2 == cmin, idx, K), axis=1, keepdims=True)
        upd = cmin < best_val
        return (jnp.where(upd, cmin, best_val),
                jnp.where(upd, carg, best_idx))

    init = (jnp.full((L, 1), jnp.inf, jnp.float32),
            jnp.zeros((L, 1), jnp.int32))
    _, best_idx = jax.lax.fori_loop(0, K // CHUNK, body, init)
    out_ref[0] = best_idx


@jax.jit
def kernel(masked_target_values, project_mat, codebook_norm):
    B, L, D = masked_target_values.shape
    K, C = codebook_norm.shape
    ct = codebook_norm.T
    csq = jnp.sum(codebook_norm * codebook_norm, axis=-1)[None, :]

    out = pl.pallas_call(
        _rpq_kernel,
        grid=(B,),
        in_specs=[
            pl.BlockSpec((1, L, D), lambda b: (b, 0, 0)),
            pl.BlockSpec((D, C), lambda b: (0, 0)),
            pl.BlockSpec((C, K), lambda b: (0, 0)),
            pl.BlockSpec((1, K), lambda b: (0, 0)),
        ],
        out_specs=pl.BlockSpec((1, L, 1), lambda b: (b, 0, 0)),
        out_shape=jax.ShapeDtypeStruct((B, L, 1), jnp.int32),
    )(masked_target_values, project_mat, ct, csq)
    return out[:, :, 0]


# speedup vs baseline: 2.1873x; 1.5710x over previous
import jax
import jax.numpy as jnp
from jax.experimental import pallas as pl
from jax.experimental.pallas import tpu as pltpu

CHUNK = 1024


def _rpq_kernel(x_ref, p_ref, baug_ref, out_ref):
    x = x_ref[0]
    t = jnp.dot(x, p_ref[...], preferred_element_type=jnp.float32)
    n = jnp.sqrt(jnp.sum(t * t, axis=-1, keepdims=True))
    nn = jnp.maximum(n, 1e-12)
    L = x.shape[0]
    K = baug_ref.shape[1]
    a = jnp.concatenate(
        [t, nn, jnp.zeros((L, 7), jnp.float32)], axis=1)

    iota = jax.lax.broadcasted_iota(jnp.int32, (L, CHUNK), 1)
    best_val = jnp.full((L, 1), jnp.inf, jnp.float32)
    best_idx = jnp.zeros((L, 1), jnp.int32)
    for c in range(K // CHUNK):
        s = jnp.dot(a, baug_ref[:, c * CHUNK:(c + 1) * CHUNK],
                    preferred_element_type=jnp.float32)
        cmin = jnp.min(s, axis=1, keepdims=True)
        carg = jnp.min(jnp.where(s == cmin, iota + c * CHUNK, K),
                       axis=1, keepdims=True)
        upd = cmin < best_val
        best_val = jnp.where(upd, cmin, best_val)
        best_idx = jnp.where(upd, carg, best_idx)
    out_ref[0] = best_idx


@jax.jit
def kernel(masked_target_values, project_mat, codebook_norm):
    B, L, D = masked_target_values.shape
    K, C = codebook_norm.shape
    csq = jnp.sum(codebook_norm * codebook_norm, axis=-1)
    baug = jnp.concatenate(
        [-2.0 * codebook_norm.T, csq[None, :],
         jnp.zeros((7, K), jnp.float32)], axis=0)

    out = pl.pallas_call(
        _rpq_kernel,
        grid=(B,),
        in_specs=[
            pl.BlockSpec((1, L, D), lambda b: (b, 0, 0)),
            pl.BlockSpec((D, C), lambda b: (0, 0)),
            pl.BlockSpec((24, K), lambda b: (0, 0)),
        ],
        out_specs=pl.BlockSpec((1, L, 1), lambda b: (b, 0, 0)),
        out_shape=jax.ShapeDtypeStruct((B, L, 1), jnp.int32),
    )(masked_target_values, project_mat, baug)
    return out[:, :, 0]
